# Initial kernel scaffold; baseline (speedup 1.0000x reference)
import jax
import jax.numpy as jnp
from jax.experimental import pallas as pl


def kernel(x, edge_index, W_self1, b_self1, W_neigh1, W_self2, b_self2, W_neigh2, W1, b1, W2, b2, W3, b3):
    raise NotImplementedError("write your pallas kernel here")


# speedup vs baseline: 3.8492x; 3.8492x over previous
import jax
import jax.numpy as jnp
from jax import lax
from jax.experimental import pallas as pl
from jax.experimental.pallas import tpu as pltpu
from jax.experimental.pallas import tpu_sc as plsc


def _make_segsum(N, D, E, with_deg):
    info = plsc.get_sparse_core_info()
    NC, NS = info.num_cores, info.num_subcores
    NW = NC * NS
    EPW = E // NW
    C = 80
    ITERS = EPW // C
    RPS = (N // NS) // 8 * 8
    TAIL = N - RPS * NS
    mesh = plsc.VectorSubcoreMesh(core_axis_name="c", subcore_axis_name="s")

    if with_deg:
        out_type = [jax.ShapeDtypeStruct((NC, N, D), jnp.float32),
                    jax.ShapeDtypeStruct((NC, N, D), jnp.float32)]
    else:
        out_type = jax.ShapeDtypeStruct((NC, N, D), jnp.float32)
    scratch = [
        pltpu.VMEM((C,), jnp.int32),
        pltpu.VMEM((C,), jnp.int32),
        pltpu.VMEM((C, D), jnp.float32),
        pltpu.VMEM_SHARED((N, D), jnp.float32),
        pltpu.SemaphoreType.DMA,
    ]

    def body(table_hbm, src_hbm, dst_hbm, zeros_hbm, ones_hbm, *refs):
        if with_deg:
            agg_out, deg_out, src_v, dst_v, rows_v, acc_sh, sem = refs
        else:
            agg_out, src_v, dst_v, rows_v, acc_sh, sem = refs
        cid = lax.axis_index("c")
        sid = lax.axis_index("s")
        wid = sid * NC + cid

        def copy_out(dst_hbm_arr):
            sl = pl.ds(sid * RPS, RPS)
            pltpu.sync_copy(acc_sh.at[sl], dst_hbm_arr.at[cid, sl])
            if TAIL:
                @pl.when(sid == NS - 1)
                def _tail():
                    tl = pl.ds(RPS * NS, TAIL)
                    pltpu.sync_copy(acc_sh.at[tl], dst_hbm_arr.at[cid, tl])

        @pl.when(sid == 0)
        def _init():
            pltpu.sync_copy(zeros_hbm, acc_sh)

        plsc.subcore_barrier()

        def step(i, carry):
            base = wid * EPW + i * C
            pltpu.sync_copy(src_hbm.at[pl.ds(base, C)], src_v)
            pltpu.sync_copy(dst_hbm.at[pl.ds(base, C)], dst_v)
            pltpu.async_copy(table_hbm.at[src_v], rows_v, sem).wait()
            pltpu.sync_copy(rows_v, acc_sh.at[dst_v], add=True)
            return carry

        lax.fori_loop(0, ITERS, step, 0)
        plsc.subcore_barrier()
        copy_out(agg_out)

        if with_deg:
            plsc.subcore_barrier()

            @pl.when(sid == 0)
            def _reinit():
                pltpu.sync_copy(zeros_hbm, acc_sh)

            pltpu.sync_copy(ones_hbm, rows_v)
            plsc.subcore_barrier()

            def dstep(i, carry):
                base = wid * EPW + i * C
                pltpu.sync_copy(dst_hbm.at[pl.ds(base, C)], dst_v)
                pltpu.sync_copy(rows_v, acc_sh.at[dst_v], add=True)
                return carry

            lax.fori_loop(0, ITERS, dstep, 0)
            plsc.subcore_barrier()
            copy_out(deg_out)

    return pl.kernel(body, mesh=mesh, out_type=out_type,
                     scratch_types=scratch)


def _make_pair_gather(N, D, E):
    info = plsc.get_sparse_core_info()
    NC, NS = info.num_cores, info.num_subcores
    NW = NC * NS
    EPW = E // NW
    C = 80
    ITERS = EPW // C
    mesh = plsc.VectorSubcoreMesh(core_axis_name="c", subcore_axis_name="s")

    out_type = [jax.ShapeDtypeStruct((E, D), jnp.float32),
                jax.ShapeDtypeStruct((E, D), jnp.float32)]
    scratch = [
        pltpu.VMEM((C,), jnp.int32),
        pltpu.VMEM((C,), jnp.int32),
        pltpu.VMEM((C, D), jnp.float32),
        pltpu.VMEM((C, D), jnp.float32),
        pltpu.SemaphoreType.DMA,
        pltpu.SemaphoreType.DMA,
    ]

    def body(table_hbm, src_hbm, dst_hbm, gu_out, gv_out,
             src_v, dst_v, ru_v, rv_v, sem_u, sem_v):
        cid = lax.axis_index("c")
        sid = lax.axis_index("s")
        wid = sid * NC + cid

        def step(i, carry):
            base = wid * EPW + i * C
            sl = pl.ds(base, C)
            pltpu.sync_copy(src_hbm.at[sl], src_v)
            pltpu.sync_copy(dst_hbm.at[sl], dst_v)
            cu = pltpu.async_copy(table_hbm.at[src_v], ru_v, sem_u)
            cv = pltpu.async_copy(table_hbm.at[dst_v], rv_v, sem_v)
            cu.wait()
            cv.wait()
            pltpu.sync_copy(ru_v, gu_out.at[sl])
            pltpu.sync_copy(rv_v, gv_out.at[sl])
            return carry

        lax.fori_loop(0, ITERS, step, 0)

    return pl.kernel(body, mesh=mesh, out_type=out_type,
                     scratch_types=scratch)


def _tc1_body(x_ref, aggp_ref, degp_ref, ws1_ref, bs1_ref, wn1_ref,
              ws2_ref, bs2_ref, wn2_ref, m2_ref, hs2_ref):
    agg = aggp_ref[0] + aggp_ref[1]
    deg = degp_ref[0, :, 0] + degp_ref[1, :, 0]
    inv = 1.0 / jnp.maximum(deg, 1.0)
    hn = agg * inv[:, None]
    h = x_ref[...] @ ws1_ref[...] + bs1_ref[...] + hn @ wn1_ref[...]
    h = jnp.maximum(h, 0.0)
    m2_ref[...] = h @ wn2_ref[...]
    hs2_ref[...] = h @ ws2_ref[...] + bs2_ref[...]


def _tc2_body(hs2_ref, aggp_ref, degp_ref, hfin_ref):
    agg = aggp_ref[0] + aggp_ref[1]
    deg = degp_ref[0, :, 0] + degp_ref[1, :, 0]
    inv = 1.0 / jnp.maximum(deg, 1.0)
    hfin_ref[...] = hs2_ref[...] + agg * inv[:, None]


def _tc3_body(gu_ref, gv_ref, w1a_ref, w1b_ref, b1_ref, w2_ref, b2_ref,
              w3_ref, b3_ref, out_ref):
    z = gu_ref[...] @ w1a_ref[...] + gv_ref[...] @ w1b_ref[...] + b1_ref[...]
    z = jnp.maximum(z, 0.0)
    z = jnp.maximum(z @ w2_ref[...] + b2_ref[...], 0.0)
    s = (z @ w3_ref[...])[:, 0:1] + b3_ref[...]
    out_ref[...] = jax.nn.sigmoid(s)


def kernel(x, edge_index, W_self1, b_self1, W_neigh1, W_self2, b_self2,
           W_neigh2, W1, b1, W2, b2, W3, b3):
    N, D_IN = x.shape
    E = edge_index.shape[1]
    D_HID = W_self1.shape[1]
    D_OUT = W_self2.shape[1]
    src = edge_index[0].astype(jnp.int32)
    dst = edge_index[1].astype(jnp.int32)

    info = plsc.get_sparse_core_info()
    NC = info.num_cores

    zeros_n = jnp.zeros((N, D_IN), jnp.float32)
    ones_c = jnp.ones((80, D_IN), jnp.float32)
    segsum1 = _make_segsum(N, D_IN, E, with_deg=True)
    aggp1, degp = segsum1(x, src, dst, zeros_n, ones_c)

    BN = 1000
    grid_n = N // BN
    tc1 = pl.pallas_call(
        _tc1_body,
        grid=(grid_n,),
        in_specs=[
            pl.BlockSpec((BN, D_IN), lambda i: (i, 0)),
            pl.BlockSpec((NC, BN, D_IN), lambda i: (0, i, 0)),
            pl.BlockSpec((NC, BN, D_IN), lambda i: (0, i, 0)),
            pl.BlockSpec((D_IN, D_HID), lambda i: (0, 0)),
            pl.BlockSpec((1, D_HID), lambda i: (0, 0)),
            pl.BlockSpec((D_IN, D_HID), lambda i: (0, 0)),
            pl.BlockSpec((D_HID, D_OUT), lambda i: (0, 0)),
            pl.BlockSpec((1, D_OUT), lambda i: (0, 0)),
            pl.BlockSpec((D_HID, D_OUT), lambda i: (0, 0)),
        ],
        out_specs=[
            pl.BlockSpec((BN, D_OUT), lambda i: (i, 0)),
            pl.BlockSpec((BN, D_OUT), lambda i: (i, 0)),
        ],
        out_shape=[
            jax.ShapeDtypeStruct((N, D_OUT), jnp.float32),
            jax.ShapeDtypeStruct((N, D_OUT), jnp.float32),
        ],
    )
    M2, Hs2 = tc1(x, aggp1, degp, W_self1, b_self1.reshape(1, -1), W_neigh1,
                  W_self2, b_self2.reshape(1, -1), W_neigh2)

    segsum2 = _make_segsum(N, D_OUT, E, with_deg=False)
    aggp2 = segsum2(M2, src, dst, zeros_n, ones_c)

    tc2 = pl.pallas_call(
        _tc2_body,
        grid=(grid_n,),
        in_specs=[
            pl.BlockSpec((BN, D_OUT), lambda i: (i, 0)),
            pl.BlockSpec((NC, BN, D_OUT), lambda i: (0, i, 0)),
            pl.BlockSpec((NC, BN, D_IN), lambda i: (0, i, 0)),
        ],
        out_specs=pl.BlockSpec((BN, D_OUT), lambda i: (i, 0)),
        out_shape=jax.ShapeDtypeStruct((N, D_OUT), jnp.float32),
    )
    hfin = tc2(Hs2, aggp2, degp)

    pair_gather = _make_pair_gather(N, D_OUT, E)
    Gu, Gv = pair_gather(hfin, src, dst)

    W1a = W1[:D_OUT]
    W1b = W1[D_OUT:]
    W3p = jnp.pad(W3, ((0, 0), (0, 127)))
    BE = 3200
    grid_e = E // BE
    tc3 = pl.pallas_call(
        _tc3_body,
        grid=(grid_e,),
        in_specs=[
            pl.BlockSpec((BE, D_OUT), lambda i: (i, 0)),
            pl.BlockSpec((BE, D_OUT), lambda i: (i, 0)),
            pl.BlockSpec((D_OUT, D_HID), lambda i: (0, 0)),
            pl.BlockSpec((D_OUT, D_HID), lambda i: (0, 0)),
            pl.BlockSpec((1, D_HID), lambda i: (0, 0)),
            pl.BlockSpec((D_HID, D_HID), lambda i: (0, 0)),
            pl.BlockSpec((1, D_HID), lambda i: (0, 0)),
            pl.BlockSpec((D_HID, 128), lambda i: (0, 0)),
            pl.BlockSpec((1, 1), lambda i: (0, 0)),
        ],
        out_specs=pl.BlockSpec((BE, 1), lambda i: (i, 0)),
        out_shape=jax.ShapeDtypeStruct((E, 1), jnp.float32),
    )
    score = tc3(Gu, Gv, W1a, W1b, b1.reshape(1, -1), W2, b2.reshape(1, -1),
                W3p, b3.reshape(1, 1))
    return score


# speedup vs baseline: 6.3472x; 1.6490x over previous
import jax
import jax.numpy as jnp
from jax import lax
from jax.experimental import pallas as pl
from jax.experimental.pallas import tpu as pltpu
from jax.experimental.pallas import tpu_sc as plsc


_C = 40
_CP = 80
_K = 5


def _make_segsum(N, D, E, with_deg):
    info = plsc.get_sparse_core_info()
    NC, NS = info.num_cores, info.num_subcores
    NW = NC * NS
    EPW = E // NW
    C, K = _C, _K
    ITERS = EPW // C
    NR = ITERS // K
    RPS = (N // NS) // 8 * 8
    TAIL = N - RPS * NS
    mesh = plsc.VectorSubcoreMesh(core_axis_name="c", subcore_axis_name="s")

    if with_deg:
        out_type = [jax.ShapeDtypeStruct((NC, N, D), jnp.float32),
                    jax.ShapeDtypeStruct((NC, N, D), jnp.float32)]
    else:
        out_type = jax.ShapeDtypeStruct((NC, N, D), jnp.float32)
    scratch = [
        pltpu.VMEM((EPW,), jnp.int32),
        pltpu.VMEM((K, C), jnp.int32),
        pltpu.VMEM((K, C, D), jnp.float32),
        pltpu.VMEM_SHARED((N, D), jnp.float32),
    ] + [pltpu.SemaphoreType.DMA] * (3 * K)

    def body(table_hbm, src_hbm, dst_hbm, zeros_hbm, ones_hbm, *refs):
        if with_deg:
            agg_out, deg_out, sidx, didx, rows, acc_sh = refs[:6]
            sems = refs[6:]
        else:
            agg_out, sidx, didx, rows, acc_sh = refs[:5]
            sems = refs[5:]
        gsems, ssems, dsems = sems[:K], sems[K:2 * K], sems[2 * K:]
        cid = lax.axis_index("c")
        sid = lax.axis_index("s")
        wid = sid * NC + cid
        ebase = wid * EPW

        def copy_out(dst_hbm_arr):
            sl = pl.ds(sid * RPS, RPS)
            pltpu.sync_copy(acc_sh.at[sl], dst_hbm_arr.at[cid, sl])
            if TAIL:
                @pl.when(sid == NS - 1)
                def _tail():
                    tl = pl.ds(RPS * NS, TAIL)
                    pltpu.sync_copy(acc_sh.at[tl], dst_hbm_arr.at[cid, tl])

        def wait_rows(buf_ref, sem):
            pltpu.make_async_copy(table_hbm.at[pl.ds(0, C)], buf_ref,
                                  sem).wait()

        def wait_idx(buf_ref, sem):
            pltpu.make_async_copy(dst_hbm.at[pl.ds(0, C)], buf_ref,
                                  sem).wait()

        def load_didx(k, c):
            pltpu.async_copy(dst_hbm.at[pl.ds(ebase + c * C, C)],
                             didx.at[k], dsems[k])

        @pl.when(sid == 0)
        def _init():
            pltpu.sync_copy(zeros_hbm, acc_sh)

        pltpu.sync_copy(src_hbm.at[pl.ds(ebase, EPW)], sidx)
        plsc.subcore_barrier()

        for k in range(K):
            load_didx(k, k)
            pltpu.async_copy(table_hbm.at[sidx.at[pl.ds(k * C, C)]],
                             rows.at[k], gsems[k])

        def rnd(r, carry):
            for k in range(K):
                wait_rows(rows.at[k], gsems[k])
                wait_idx(didx.at[k], dsems[k])
                pltpu.async_copy(rows.at[k], acc_sh.at[didx.at[k]],
                                 ssems[k], add=True)
            for k in range(K):
                c = r * K + k
                wait_rows(rows.at[k], ssems[k])

                @pl.when(r < NR - 1)
                def _refire():
                    load_didx(k, c + K)
                    pltpu.async_copy(
                        table_hbm.at[sidx.at[pl.ds((c + K) * C, C)]],
                        rows.at[k], gsems[k])
            return carry

        lax.fori_loop(0, NR, rnd, 0)
        plsc.subcore_barrier()
        copy_out(agg_out)

        if with_deg:
            plsc.subcore_barrier()

            @pl.when(sid == 0)
            def _reinit():
                pltpu.sync_copy(zeros_hbm, acc_sh)

            pltpu.sync_copy(ones_hbm, rows.at[0])
            plsc.subcore_barrier()

            for k in range(K):
                load_didx(k, k)

            def drnd(r, carry):
                for k in range(K):
                    wait_idx(didx.at[k], dsems[k])
                    pltpu.async_copy(rows.at[0], acc_sh.at[didx.at[k]],
                                     ssems[k], add=True)
                for k in range(K):
                    c = r * K + k
                    wait_rows(rows.at[0], ssems[k])

                    @pl.when(r < NR - 1)
                    def _refire():
                        load_didx(k, c + K)
                return carry

            lax.fori_loop(0, NR, drnd, 0)
            plsc.subcore_barrier()
            copy_out(deg_out)

    return pl.kernel(body, mesh=mesh, out_type=out_type,
                     scratch_types=scratch)


def _make_pair_gather(N, D, E):
    info = plsc.get_sparse_core_info()
    NC, NS = info.num_cores, info.num_subcores
    NW = NC * NS
    EPW = E // NW
    C, K = _CP, _K
    ITERS = EPW // C
    NR = ITERS // K
    mesh = plsc.VectorSubcoreMesh(core_axis_name="c", subcore_axis_name="s")

    out_type = [jax.ShapeDtypeStruct((E, D), jnp.float32),
                jax.ShapeDtypeStruct((E, D), jnp.float32)]
    scratch = [
        pltpu.VMEM((EPW,), jnp.int32),
        pltpu.VMEM((EPW,), jnp.int32),
        pltpu.VMEM((K, C, D), jnp.float32),
        pltpu.VMEM((K, C, D), jnp.float32),
    ] + [pltpu.SemaphoreType.DMA] * (4 * K)

    def body(table_hbm, src_hbm, dst_hbm, gu_out, gv_out, *refs):
        sidx, didx, ru, rv = refs[:4]
        gusems = refs[4:4 + K]
        gvsems = refs[4 + K:4 + 2 * K]
        wusems = refs[4 + 2 * K:4 + 3 * K]
        wvsems = refs[4 + 3 * K:]
        cid = lax.axis_index("c")
        sid = lax.axis_index("s")
        wid = sid * NC + cid
        ebase = wid * EPW

        def wait(buf_ref, sem):
            pltpu.make_async_copy(table_hbm.at[pl.ds(0, C)], buf_ref,
                                  sem).wait()

        pltpu.sync_copy(src_hbm.at[pl.ds(ebase, EPW)], sidx)
        pltpu.sync_copy(dst_hbm.at[pl.ds(ebase, EPW)], didx)

        def fire(k, c):
            pltpu.async_copy(table_hbm.at[sidx.at[pl.ds(c * C, C)]],
                             ru.at[k], gusems[k])
            pltpu.async_copy(table_hbm.at[didx.at[pl.ds(c * C, C)]],
                             rv.at[k], gvsems[k])

        for k in range(K):
            fire(k, k)

        def rnd(r, carry):
            for k in range(K):
                c = r * K + k
                base = ebase + c * C
                wait(ru.at[k], gusems[k])
                pltpu.async_copy(ru.at[k], gu_out.at[pl.ds(base, C)],
                                 wusems[k])
                wait(rv.at[k], gvsems[k])
                pltpu.async_copy(rv.at[k], gv_out.at[pl.ds(base, C)],
                                 wvsems[k])
            for k in range(K):
                c = r * K + k
                wait(ru.at[k], wusems[k])
                wait(rv.at[k], wvsems[k])

                @pl.when(r < NR - 1)
                def _refire():
                    fire(k, c + K)
            return carry

        lax.fori_loop(0, NR, rnd, 0)

    return pl.kernel(body, mesh=mesh, out_type=out_type,
                     scratch_types=scratch)


def _tc1_body(x_ref, aggp_ref, degp_ref, ws1_ref, bs1_ref, wn1_ref,
              ws2_ref, bs2_ref, wn2_ref, m2_ref, hs2_ref):
    agg = aggp_ref[0] + aggp_ref[1]
    deg = degp_ref[0, :, 0] + degp_ref[1, :, 0]
    inv = 1.0 / jnp.maximum(deg, 1.0)
    hn = agg * inv[:, None]
    h = x_ref[...] @ ws1_ref[...] + bs1_ref[...] + hn @ wn1_ref[...]
    h = jnp.maximum(h, 0.0)
    m2_ref[...] = h @ wn2_ref[...]
    hs2_ref[...] = h @ ws2_ref[...] + bs2_ref[...]


def _tc2_body(hs2_ref, aggp_ref, degp_ref, hfin_ref):
    agg = aggp_ref[0] + aggp_ref[1]
    deg = degp_ref[0, :, 0] + degp_ref[1, :, 0]
    inv = 1.0 / jnp.maximum(deg, 1.0)
    hfin_ref[...] = hs2_ref[...] + agg * inv[:, None]


def _tc3_body(gu_ref, gv_ref, w1a_ref, w1b_ref, b1_ref, w2_ref, b2_ref,
              w3_ref, b3_ref, out_ref):
    z = gu_ref[...] @ w1a_ref[...] + gv_ref[...] @ w1b_ref[...] + b1_ref[...]
    z = jnp.maximum(z, 0.0)
    z = jnp.maximum(z @ w2_ref[...] + b2_ref[...], 0.0)
    s = (z @ w3_ref[...])[:, 0:1] + b3_ref[...]
    out_ref[...] = jax.nn.sigmoid(s)


def kernel(x, edge_index, W_self1, b_self1, W_neigh1, W_self2, b_self2,
           W_neigh2, W1, b1, W2, b2, W3, b3):
    N, D_IN = x.shape
    E = edge_index.shape[1]
    D_HID = W_self1.shape[1]
    D_OUT = W_self2.shape[1]
    src = edge_index[0].astype(jnp.int32)
    dst = edge_index[1].astype(jnp.int32)

    info = plsc.get_sparse_core_info()
    NC = info.num_cores

    zeros_n = jnp.zeros((N, D_IN), jnp.float32)
    ones_c = jnp.ones((_C, D_IN), jnp.float32)
    segsum1 = _make_segsum(N, D_IN, E, with_deg=True)
    aggp1, degp = segsum1(x, src, dst, zeros_n, ones_c)

    BN = 1000
    grid_n = N // BN
    tc1 = pl.pallas_call(
        _tc1_body,
        grid=(grid_n,),
        in_specs=[
            pl.BlockSpec((BN, D_IN), lambda i: (i, 0)),
            pl.BlockSpec((NC, BN, D_IN), lambda i: (0, i, 0)),
            pl.BlockSpec((NC, BN, D_IN), lambda i: (0, i, 0)),
            pl.BlockSpec((D_IN, D_HID), lambda i: (0, 0)),
            pl.BlockSpec((1, D_HID), lambda i: (0, 0)),
            pl.BlockSpec((D_IN, D_HID), lambda i: (0, 0)),
            pl.BlockSpec((D_HID, D_OUT), lambda i: (0, 0)),
            pl.BlockSpec((1, D_OUT), lambda i: (0, 0)),
            pl.BlockSpec((D_HID, D_OUT), lambda i: (0, 0)),
        ],
        out_specs=[
            pl.BlockSpec((BN, D_OUT), lambda i: (i, 0)),
            pl.BlockSpec((BN, D_OUT), lambda i: (i, 0)),
        ],
        out_shape=[
            jax.ShapeDtypeStruct((N, D_OUT), jnp.float32),
            jax.ShapeDtypeStruct((N, D_OUT), jnp.float32),
        ],
    )
    M2, Hs2 = tc1(x, aggp1, degp, W_self1, b_self1.reshape(1, -1), W_neigh1,
                  W_self2, b_self2.reshape(1, -1), W_neigh2)

    segsum2 = _make_segsum(N, D_OUT, E, with_deg=False)
    aggp2 = segsum2(M2, src, dst, zeros_n, ones_c)

    tc2 = pl.pallas_call(
        _tc2_body,
        grid=(grid_n,),
        in_specs=[
            pl.BlockSpec((BN, D_OUT), lambda i: (i, 0)),
            pl.BlockSpec((NC, BN, D_OUT), lambda i: (0, i, 0)),
            pl.BlockSpec((NC, BN, D_IN), lambda i: (0, i, 0)),
        ],
        out_specs=pl.BlockSpec((BN, D_OUT), lambda i: (i, 0)),
        out_shape=jax.ShapeDtypeStruct((N, D_OUT), jnp.float32),
    )
    hfin = tc2(Hs2, aggp2, degp)

    pair_gather = _make_pair_gather(N, D_OUT, E)
    Gu, Gv = pair_gather(hfin, src, dst)

    W1a = W1[:D_OUT]
    W1b = W1[D_OUT:]
    W3p = jnp.pad(W3, ((0, 0), (0, 127)))
    BE = 3200
    grid_e = E // BE
    tc3 = pl.pallas_call(
        _tc3_body,
        grid=(grid_e,),
        in_specs=[
            pl.BlockSpec((BE, D_OUT), lambda i: (i, 0)),
            pl.BlockSpec((BE, D_OUT), lambda i: (i, 0)),
            pl.BlockSpec((D_OUT, D_HID), lambda i: (0, 0)),
            pl.BlockSpec((D_OUT, D_HID), lambda i: (0, 0)),
            pl.BlockSpec((1, D_HID), lambda i: (0, 0)),
            pl.BlockSpec((D_HID, D_HID), lambda i: (0, 0)),
            pl.BlockSpec((1, D_HID), lambda i: (0, 0)),
            pl.BlockSpec((D_HID, 128), lambda i: (0, 0)),
            pl.BlockSpec((1, 1), lambda i: (0, 0)),
        ],
        out_specs=pl.BlockSpec((BE, 1), lambda i: (i, 0)),
        out_shape=jax.ShapeDtypeStruct((E, 1), jnp.float32),
    )
    score = tc3(Gu, Gv, W1a, W1b, b1.reshape(1, -1), W2, b2.reshape(1, -1),
                W3p, b3.reshape(1, 1))
    return score


# speedup vs baseline: 6.5623x; 1.0339x over previous
import jax
import jax.numpy as jnp
from jax import lax
from jax.experimental import pallas as pl
from jax.experimental.pallas import tpu as pltpu
from jax.experimental.pallas import tpu_sc as plsc


_C = 40
_CP = 80
_K = 5


def _make_segsum(N, D, E, with_deg):
    info = plsc.get_sparse_core_info()
    NC, NS = info.num_cores, info.num_subcores
    NW = NC * NS
    EPW = E // NW
    C, K = _C, _K
    ITERS = EPW // C
    NR = ITERS // K
    RPS = (N // NS) // 8 * 8
    TAIL = N - RPS * NS
    mesh = plsc.VectorSubcoreMesh(core_axis_name="c", subcore_axis_name="s")

    if with_deg:
        out_type = [jax.ShapeDtypeStruct((NC, N, D), jnp.float32),
                    jax.ShapeDtypeStruct((NC, N, D), jnp.float32)]
    else:
        out_type = jax.ShapeDtypeStruct((NC, N, D), jnp.float32)
    scratch = [
        pltpu.VMEM((EPW,), jnp.int32),
        pltpu.VMEM((K, C), jnp.int32),
        pltpu.VMEM((K, C, D), jnp.float32),
        pltpu.VMEM_SHARED((N, D), jnp.float32),
    ] + [pltpu.SemaphoreType.DMA] * (3 * K)

    def body(table_hbm, src_hbm, dst_hbm, zeros_hbm, ones_hbm, *refs):
        if with_deg:
            agg_out, deg_out, sidx, didx, rows, acc_sh = refs[:6]
            sems = refs[6:]
        else:
            agg_out, sidx, didx, rows, acc_sh = refs[:5]
            sems = refs[5:]
        gsems, ssems, dsems = sems[:K], sems[K:2 * K], sems[2 * K:]
        cid = lax.axis_index("c")
        sid = lax.axis_index("s")
        wid = sid * NC + cid
        ebase = wid * EPW

        def copy_out(dst_hbm_arr):
            sl = pl.ds(sid * RPS, RPS)
            pltpu.sync_copy(acc_sh.at[sl], dst_hbm_arr.at[cid, sl])
            if TAIL:
                @pl.when(sid == NS - 1)
                def _tail():
                    tl = pl.ds(RPS * NS, TAIL)
                    pltpu.sync_copy(acc_sh.at[tl], dst_hbm_arr.at[cid, tl])

        def wait_rows(buf_ref, sem):
            pltpu.make_async_copy(table_hbm.at[pl.ds(0, C)], buf_ref,
                                  sem).wait()

        def wait_idx(buf_ref, sem):
            pltpu.make_async_copy(dst_hbm.at[pl.ds(0, C)], buf_ref,
                                  sem).wait()

        def load_didx(k, c):
            pltpu.async_copy(dst_hbm.at[pl.ds(ebase + c * C, C)],
                             didx.at[k], dsems[k])

        @pl.when(sid == 0)
        def _init():
            pltpu.sync_copy(zeros_hbm, acc_sh)

        pltpu.sync_copy(src_hbm.at[pl.ds(ebase, EPW)], sidx)
        plsc.subcore_barrier()

        for k in range(K):
            load_didx(k, k)
            pltpu.async_copy(table_hbm.at[sidx.at[pl.ds(k * C, C)]],
                             rows.at[k], gsems[k])

        def rnd(r, carry):
            for k in range(K):
                wait_rows(rows.at[k], gsems[k])
                wait_idx(didx.at[k], dsems[k])
                pltpu.async_copy(rows.at[k], acc_sh.at[didx.at[k]],
                                 ssems[k], add=True)
            for k in range(K):
                c = r * K + k
                wait_rows(rows.at[k], ssems[k])

                @pl.when(r < NR - 1)
                def _refire():
                    load_didx(k, c + K)
                    pltpu.async_copy(
                        table_hbm.at[sidx.at[pl.ds((c + K) * C, C)]],
                        rows.at[k], gsems[k])
            return carry

        lax.fori_loop(0, NR, rnd, 0)
        plsc.subcore_barrier()
        copy_out(agg_out)

        if with_deg:
            plsc.subcore_barrier()

            @pl.when(sid == 0)
            def _reinit():
                pltpu.sync_copy(zeros_hbm, acc_sh)

            pltpu.sync_copy(ones_hbm, rows.at[0])
            plsc.subcore_barrier()

            for k in range(K):
                load_didx(k, k)

            def drnd(r, carry):
                for k in range(K):
                    wait_idx(didx.at[k], dsems[k])
                    pltpu.async_copy(rows.at[0], acc_sh.at[didx.at[k]],
                                     ssems[k], add=True)
                for k in range(K):
                    c = r * K + k
                    wait_rows(rows.at[0], ssems[k])

                    @pl.when(r < NR - 1)
                    def _refire():
                        load_didx(k, c + K)
                return carry

            lax.fori_loop(0, NR, drnd, 0)
            plsc.subcore_barrier()
            copy_out(deg_out)

    return pl.kernel(body, mesh=mesh, out_type=out_type,
                     scratch_types=scratch)


def _make_pair_gather(N, D, E, offset=0):
    info = plsc.get_sparse_core_info()
    NC, NS = info.num_cores, info.num_subcores
    NW = NC * NS
    EPW = E // NW
    C, K = _CP, _K
    ITERS = EPW // C
    NR = ITERS // K
    mesh = plsc.VectorSubcoreMesh(core_axis_name="c", subcore_axis_name="s")

    out_type = [jax.ShapeDtypeStruct((E, D), jnp.float32),
                jax.ShapeDtypeStruct((E, D), jnp.float32)]
    scratch = [
        pltpu.VMEM((EPW,), jnp.int32),
        pltpu.VMEM((EPW,), jnp.int32),
        pltpu.VMEM((K, C, D), jnp.float32),
        pltpu.VMEM((K, C, D), jnp.float32),
    ] + [pltpu.SemaphoreType.DMA] * (4 * K)

    def body(table_hbm, src_hbm, dst_hbm, gu_out, gv_out, *refs):
        sidx, didx, ru, rv = refs[:4]
        gusems = refs[4:4 + K]
        gvsems = refs[4 + K:4 + 2 * K]
        wusems = refs[4 + 2 * K:4 + 3 * K]
        wvsems = refs[4 + 3 * K:]
        cid = lax.axis_index("c")
        sid = lax.axis_index("s")
        wid = sid * NC + cid
        ebase = wid * EPW

        def wait(buf_ref, sem):
            pltpu.make_async_copy(table_hbm.at[pl.ds(0, C)], buf_ref,
                                  sem).wait()

        pltpu.sync_copy(src_hbm.at[pl.ds(offset + ebase, EPW)], sidx)
        pltpu.sync_copy(dst_hbm.at[pl.ds(offset + ebase, EPW)], didx)

        def fire(k, c):
            pltpu.async_copy(table_hbm.at[sidx.at[pl.ds(c * C, C)]],
                             ru.at[k], gusems[k])
            pltpu.async_copy(table_hbm.at[didx.at[pl.ds(c * C, C)]],
                             rv.at[k], gvsems[k])

        for k in range(K):
            fire(k, k)

        def rnd(r, carry):
            for k in range(K):
                c = r * K + k
                base = ebase + c * C
                wait(ru.at[k], gusems[k])
                pltpu.async_copy(ru.at[k], gu_out.at[pl.ds(base, C)],
                                 wusems[k])
                wait(rv.at[k], gvsems[k])
                pltpu.async_copy(rv.at[k], gv_out.at[pl.ds(base, C)],
                                 wvsems[k])
            for k in range(K):
                c = r * K + k
                wait(ru.at[k], wusems[k])
                wait(rv.at[k], wvsems[k])

                @pl.when(r < NR - 1)
                def _refire():
                    fire(k, c + K)
            return carry

        lax.fori_loop(0, NR, rnd, 0)

    return pl.kernel(body, mesh=mesh, out_type=out_type,
                     scratch_types=scratch)


def _tc1_body(x_ref, aggp_ref, degp_ref, ws1_ref, bs1_ref, wn1_ref,
              ws2_ref, bs2_ref, wn2_ref, m2_ref, hs2_ref):
    agg = aggp_ref[0] + aggp_ref[1]
    deg = degp_ref[0, :, 0] + degp_ref[1, :, 0]
    inv = 1.0 / jnp.maximum(deg, 1.0)
    hn = agg * inv[:, None]
    h = x_ref[...] @ ws1_ref[...] + bs1_ref[...] + hn @ wn1_ref[...]
    h = jnp.maximum(h, 0.0)
    m2_ref[...] = h @ wn2_ref[...]
    hs2_ref[...] = h @ ws2_ref[...] + bs2_ref[...]


def _tc2_body(hs2_ref, aggp_ref, degp_ref, hfin_ref):
    agg = aggp_ref[0] + aggp_ref[1]
    deg = degp_ref[0, :, 0] + degp_ref[1, :, 0]
    inv = 1.0 / jnp.maximum(deg, 1.0)
    hfin_ref[...] = hs2_ref[...] + agg * inv[:, None]


def _tc3_body(gu_ref, gv_ref, w1a_ref, w1b_ref, b1_ref, w2_ref, b2_ref,
              w3_ref, b3_ref, out_ref):
    z = gu_ref[...] @ w1a_ref[...] + gv_ref[...] @ w1b_ref[...] + b1_ref[...]
    z = jnp.maximum(z, 0.0)
    z = jnp.maximum(z @ w2_ref[...] + b2_ref[...], 0.0)
    s = jnp.sum(z * w3_ref[...], axis=1, keepdims=True) + b3_ref[...]
    out_ref[...] = jax.nn.sigmoid(s)


def kernel(x, edge_index, W_self1, b_self1, W_neigh1, W_self2, b_self2,
           W_neigh2, W1, b1, W2, b2, W3, b3):
    N, D_IN = x.shape
    E = edge_index.shape[1]
    D_HID = W_self1.shape[1]
    D_OUT = W_self2.shape[1]
    src = edge_index[0].astype(jnp.int32)
    dst = edge_index[1].astype(jnp.int32)

    info = plsc.get_sparse_core_info()
    NC = info.num_cores

    zeros_n = jnp.zeros((N, D_IN), jnp.float32)
    ones_c = jnp.ones((_C, D_IN), jnp.float32)
    segsum1 = _make_segsum(N, D_IN, E, with_deg=True)
    aggp1, degp = segsum1(x, src, dst, zeros_n, ones_c)

    BN = 1000
    grid_n = N // BN
    tc1 = pl.pallas_call(
        _tc1_body,
        grid=(grid_n,),
        in_specs=[
            pl.BlockSpec((BN, D_IN), lambda i: (i, 0)),
            pl.BlockSpec((NC, BN, D_IN), lambda i: (0, i, 0)),
            pl.BlockSpec((NC, BN, D_IN), lambda i: (0, i, 0)),
            pl.BlockSpec((D_IN, D_HID), lambda i: (0, 0)),
            pl.BlockSpec((1, D_HID), lambda i: (0, 0)),
            pl.BlockSpec((D_IN, D_HID), lambda i: (0, 0)),
            pl.BlockSpec((D_HID, D_OUT), lambda i: (0, 0)),
            pl.BlockSpec((1, D_OUT), lambda i: (0, 0)),
            pl.BlockSpec((D_HID, D_OUT), lambda i: (0, 0)),
        ],
        out_specs=[
            pl.BlockSpec((BN, D_OUT), lambda i: (i, 0)),
            pl.BlockSpec((BN, D_OUT), lambda i: (i, 0)),
        ],
        out_shape=[
            jax.ShapeDtypeStruct((N, D_OUT), jnp.float32),
            jax.ShapeDtypeStruct((N, D_OUT), jnp.float32),
        ],
    )
    M2, Hs2 = tc1(x, aggp1, degp, W_self1, b_self1.reshape(1, -1), W_neigh1,
                  W_self2, b_self2.reshape(1, -1), W_neigh2)

    segsum2 = _make_segsum(N, D_OUT, E, with_deg=False)
    aggp2 = segsum2(M2, src, dst, zeros_n, ones_c)

    tc2 = pl.pallas_call(
        _tc2_body,
        grid=(grid_n,),
        in_specs=[
            pl.BlockSpec((BN, D_OUT), lambda i: (i, 0)),
            pl.BlockSpec((NC, BN, D_OUT), lambda i: (0, i, 0)),
            pl.BlockSpec((NC, BN, D_IN), lambda i: (0, i, 0)),
        ],
        out_specs=pl.BlockSpec((BN, D_OUT), lambda i: (i, 0)),
        out_shape=jax.ShapeDtypeStruct((N, D_OUT), jnp.float32),
    )
    hfin = tc2(Hs2, aggp2, degp)

    W1a = W1[:D_OUT]
    W1b = W1[D_OUT:]
    w3r = W3.reshape(1, -1)
    S = 5
    E_S = E // S
    BE = 3200
    grid_e = E_S // BE
    tc3 = pl.pallas_call(
        _tc3_body,
        grid=(grid_e,),
        in_specs=[
            pl.BlockSpec((BE, D_OUT), lambda i: (i, 0)),
            pl.BlockSpec((BE, D_OUT), lambda i: (i, 0)),
            pl.BlockSpec((D_OUT, D_HID), lambda i: (0, 0)),
            pl.BlockSpec((D_OUT, D_HID), lambda i: (0, 0)),
            pl.BlockSpec((1, D_HID), lambda i: (0, 0)),
            pl.BlockSpec((D_HID, D_HID), lambda i: (0, 0)),
            pl.BlockSpec((1, D_HID), lambda i: (0, 0)),
            pl.BlockSpec((1, D_HID), lambda i: (0, 0)),
            pl.BlockSpec((1, 1), lambda i: (0, 0)),
        ],
        out_specs=pl.BlockSpec((BE, 1), lambda i: (i, 0)),
        out_shape=jax.ShapeDtypeStruct((E_S, 1), jnp.float32),
    )
    scores = []
    table = hfin
    for s in range(S):
        pg = _make_pair_gather(N, D_OUT, E_S, offset=s * E_S)
        Gu, Gv = pg(table, src, dst)
        table = hfin + 0.0 * Gu[0:1, 0:1]
        scores.append(tc3(Gu, Gv, W1a, W1b, b1.reshape(1, -1), W2,
                          b2.reshape(1, -1), w3r, b3.reshape(1, 1)))
    return jnp.concatenate(scores, axis=0)


# speedup vs baseline: 6.9065x; 1.0524x over previous
import jax
import jax.numpy as jnp
from jax import lax
from jax.experimental import pallas as pl
from jax.experimental.pallas import tpu as pltpu
from jax.experimental.pallas import tpu_sc as plsc


_C = 40
_CP = 80
_K = 5


def _make_segsum(N, D, E, with_deg):
    info = plsc.get_sparse_core_info()
    NC, NS = info.num_cores, info.num_subcores
    NW = NC * NS
    EPW = E // NW
    C, K = _C, _K
    ITERS = EPW // C
    NR = ITERS // K
    RPS = (N // NS) // 8 * 8
    TAIL = N - RPS * NS
    mesh = plsc.VectorSubcoreMesh(core_axis_name="c", subcore_axis_name="s")

    if with_deg:
        out_type = [jax.ShapeDtypeStruct((NC, N, D), jnp.float32),
                    jax.ShapeDtypeStruct((NC, N, D), jnp.float32)]
    else:
        out_type = jax.ShapeDtypeStruct((NC, N, D), jnp.float32)
    scratch = [
        pltpu.VMEM((EPW,), jnp.int32),
        pltpu.VMEM((K, C), jnp.int32),
        pltpu.VMEM((K, C, D), jnp.float32),
        pltpu.VMEM_SHARED((N, D), jnp.float32),
    ] + [pltpu.SemaphoreType.DMA] * (3 * K)

    def body(table_hbm, src_hbm, dst_hbm, zeros_hbm, ones_hbm, *refs):
        if with_deg:
            agg_out, deg_out, sidx, didx, rows, acc_sh = refs[:6]
            sems = refs[6:]
        else:
            agg_out, sidx, didx, rows, acc_sh = refs[:5]
            sems = refs[5:]
        gsems, ssems, dsems = sems[:K], sems[K:2 * K], sems[2 * K:]
        cid = lax.axis_index("c")
        sid = lax.axis_index("s")
        wid = sid * NC + cid
        ebase = wid * EPW

        def copy_out(dst_hbm_arr):
            sl = pl.ds(sid * RPS, RPS)
            pltpu.sync_copy(acc_sh.at[sl], dst_hbm_arr.at[cid, sl])
            if TAIL:
                @pl.when(sid == NS - 1)
                def _tail():
                    tl = pl.ds(RPS * NS, TAIL)
                    pltpu.sync_copy(acc_sh.at[tl], dst_hbm_arr.at[cid, tl])

        def wait_rows(buf_ref, sem):
            pltpu.make_async_copy(table_hbm.at[pl.ds(0, C)], buf_ref,
                                  sem).wait()

        def wait_idx(buf_ref, sem):
            pltpu.make_async_copy(dst_hbm.at[pl.ds(0, C)], buf_ref,
                                  sem).wait()

        def load_didx(k, c):
            pltpu.async_copy(dst_hbm.at[pl.ds(ebase + c * C, C)],
                             didx.at[k], dsems[k])

        @pl.when(sid == 0)
        def _init():
            pltpu.sync_copy(zeros_hbm, acc_sh)

        pltpu.sync_copy(src_hbm.at[pl.ds(ebase, EPW)], sidx)
        plsc.subcore_barrier()

        for k in range(K):
            load_didx(k, k)
            pltpu.async_copy(table_hbm.at[sidx.at[pl.ds(k * C, C)]],
                             rows.at[k], gsems[k])

        def rnd(r, carry):
            for k in range(K):
                wait_rows(rows.at[k], gsems[k])
                wait_idx(didx.at[k], dsems[k])
                pltpu.async_copy(rows.at[k], acc_sh.at[didx.at[k]],
                                 ssems[k], add=True)
            for k in range(K):
                c = r * K + k
                wait_rows(rows.at[k], ssems[k])

                @pl.when(r < NR - 1)
                def _refire():
                    load_didx(k, c + K)
                    pltpu.async_copy(
                        table_hbm.at[sidx.at[pl.ds((c + K) * C, C)]],
                        rows.at[k], gsems[k])
            return carry

        lax.fori_loop(0, NR, rnd, 0)
        plsc.subcore_barrier()
        copy_out(agg_out)

        if with_deg:
            plsc.subcore_barrier()

            @pl.when(sid == 0)
            def _reinit():
                pltpu.sync_copy(zeros_hbm, acc_sh)

            pltpu.sync_copy(ones_hbm, rows.at[0])
            plsc.subcore_barrier()

            for k in range(K):
                load_didx(k, k)

            def drnd(r, carry):
                for k in range(K):
                    wait_idx(didx.at[k], dsems[k])
                    pltpu.async_copy(rows.at[0], acc_sh.at[didx.at[k]],
                                     ssems[k], add=True)
                for k in range(K):
                    c = r * K + k
                    wait_rows(rows.at[0], ssems[k])

                    @pl.when(r < NR - 1)
                    def _refire():
                        load_didx(k, c + K)
                return carry

            lax.fori_loop(0, NR, drnd, 0)
            plsc.subcore_barrier()
            copy_out(deg_out)

    return pl.kernel(body, mesh=mesh, out_type=out_type,
                     scratch_types=scratch)


def _make_pair_gather(N, D, E, offset=0):
    info = plsc.get_sparse_core_info()
    NC, NS = info.num_cores, info.num_subcores
    NW = NC * NS
    EPW = E // NW
    C, K = _CP, _K
    ITERS = EPW // C
    NR = ITERS // K
    mesh = plsc.VectorSubcoreMesh(core_axis_name="c", subcore_axis_name="s")

    out_type = [jax.ShapeDtypeStruct((E, D), jnp.float32),
                jax.ShapeDtypeStruct((E, D), jnp.float32)]
    scratch = [
        pltpu.VMEM((EPW,), jnp.int32),
        pltpu.VMEM((EPW,), jnp.int32),
        pltpu.VMEM((K, C, D), jnp.float32),
        pltpu.VMEM((K, C, D), jnp.float32),
    ] + [pltpu.SemaphoreType.DMA] * (4 * K)

    def body(table_hbm, src_hbm, dst_hbm, gu_out, gv_out, *refs):
        sidx, didx, ru, rv = refs[:4]
        gusems = refs[4:4 + K]
        gvsems = refs[4 + K:4 + 2 * K]
        wusems = refs[4 + 2 * K:4 + 3 * K]
        wvsems = refs[4 + 3 * K:]
        cid = lax.axis_index("c")
        sid = lax.axis_index("s")
        wid = sid * NC + cid
        ebase = wid * EPW

        def wait(buf_ref, sem):
            pltpu.make_async_copy(table_hbm.at[pl.ds(0, C)], buf_ref,
                                  sem).wait()

        pltpu.sync_copy(src_hbm.at[pl.ds(offset + ebase, EPW)], sidx)
        pltpu.sync_copy(dst_hbm.at[pl.ds(offset + ebase, EPW)], didx)

        def fire(k, c):
            pltpu.async_copy(table_hbm.at[sidx.at[pl.ds(c * C, C)]],
                             ru.at[k], gusems[k])
            pltpu.async_copy(table_hbm.at[didx.at[pl.ds(c * C, C)]],
                             rv.at[k], gvsems[k])

        for k in range(K):
            fire(k, k)

        def rnd(r, carry):
            for k in range(K):
                c = r * K + k
                base = ebase + c * C
                wait(ru.at[k], gusems[k])
                pltpu.async_copy(ru.at[k], gu_out.at[pl.ds(base, C)],
                                 wusems[k])
                wait(rv.at[k], gvsems[k])
                pltpu.async_copy(rv.at[k], gv_out.at[pl.ds(base, C)],
                                 wvsems[k])
            for k in range(K):
                c = r * K + k
                wait(ru.at[k], wusems[k])
                wait(rv.at[k], wvsems[k])

                @pl.when(r < NR - 1)
                def _refire():
                    fire(k, c + K)
            return carry

        lax.fori_loop(0, NR, rnd, 0)

    return pl.kernel(body, mesh=mesh, out_type=out_type,
                     scratch_types=scratch)


def _tc1_body(x_ref, aggp_ref, degp_ref, ws1_ref, bs1_ref, wn1_ref,
              ws2_ref, bs2_ref, wn2_ref, m2_ref, hs2_ref):
    agg = aggp_ref[0] + aggp_ref[1]
    deg = degp_ref[0, :, 0] + degp_ref[1, :, 0]
    inv = 1.0 / jnp.maximum(deg, 1.0)
    hn = agg * inv[:, None]
    h = x_ref[...] @ ws1_ref[...] + bs1_ref[...] + hn @ wn1_ref[...]
    h = jnp.maximum(h, 0.0)
    m2_ref[...] = h @ wn2_ref[...]
    hs2_ref[...] = h @ ws2_ref[...] + bs2_ref[...]


def _tc2_body(hs2_ref, aggp_ref, degp_ref, hfin_ref):
    agg = aggp_ref[0] + aggp_ref[1]
    deg = degp_ref[0, :, 0] + degp_ref[1, :, 0]
    inv = 1.0 / jnp.maximum(deg, 1.0)
    hfin_ref[...] = hs2_ref[...] + agg * inv[:, None]


def _tc3_body(gu_ref, gv_ref, w1a_ref, w1b_ref, b1_ref, w2_ref, b2_ref,
              w3_ref, b3_ref, out_ref):
    z = gu_ref[...] @ w1a_ref[...] + gv_ref[...] @ w1b_ref[...] + b1_ref[...]
    z = jnp.maximum(z, 0.0)
    z = jnp.maximum(z @ w2_ref[...] + b2_ref[...], 0.0)
    s = jnp.sum(z * w3_ref[...], axis=1, keepdims=True) + b3_ref[...]
    out_ref[...] = jax.nn.sigmoid(s)


def kernel(x, edge_index, W_self1, b_self1, W_neigh1, W_self2, b_self2,
           W_neigh2, W1, b1, W2, b2, W3, b3):
    N, D_IN = x.shape
    E = edge_index.shape[1]
    D_HID = W_self1.shape[1]
    D_OUT = W_self2.shape[1]
    src = edge_index[0].astype(jnp.int32)
    dst = edge_index[1].astype(jnp.int32)

    info = plsc.get_sparse_core_info()
    NC = info.num_cores

    zeros_n = jnp.zeros((N, D_IN), jnp.float32)
    ones_c = jnp.ones((_C, D_IN), jnp.float32)
    segsum1 = _make_segsum(N, D_IN, E, with_deg=True)
    aggp1, degp = segsum1(x, src, dst, zeros_n, ones_c)

    BN = 1000
    grid_n = N // BN
    tc1 = pl.pallas_call(
        _tc1_body,
        grid=(grid_n,),
        in_specs=[
            pl.BlockSpec((BN, D_IN), lambda i: (i, 0)),
            pl.BlockSpec((NC, BN, D_IN), lambda i: (0, i, 0)),
            pl.BlockSpec((NC, BN, D_IN), lambda i: (0, i, 0)),
            pl.BlockSpec((D_IN, D_HID), lambda i: (0, 0)),
            pl.BlockSpec((1, D_HID), lambda i: (0, 0)),
            pl.BlockSpec((D_IN, D_HID), lambda i: (0, 0)),
            pl.BlockSpec((D_HID, D_OUT), lambda i: (0, 0)),
            pl.BlockSpec((1, D_OUT), lambda i: (0, 0)),
            pl.BlockSpec((D_HID, D_OUT), lambda i: (0, 0)),
        ],
        out_specs=[
            pl.BlockSpec((BN, D_OUT), lambda i: (i, 0)),
            pl.BlockSpec((BN, D_OUT), lambda i: (i, 0)),
        ],
        out_shape=[
            jax.ShapeDtypeStruct((N, D_OUT), jnp.float32),
            jax.ShapeDtypeStruct((N, D_OUT), jnp.float32),
        ],
    )
    M2, Hs2 = tc1(x, aggp1, degp, W_self1, b_self1.reshape(1, -1), W_neigh1,
                  W_self2, b_self2.reshape(1, -1), W_neigh2)

    segsum2 = _make_segsum(N, D_OUT, E, with_deg=False)
    aggp2 = segsum2(M2, src, dst, zeros_n, ones_c)

    tc2 = pl.pallas_call(
        _tc2_body,
        grid=(grid_n,),
        in_specs=[
            pl.BlockSpec((BN, D_OUT), lambda i: (i, 0)),
            pl.BlockSpec((NC, BN, D_OUT), lambda i: (0, i, 0)),
            pl.BlockSpec((NC, BN, D_IN), lambda i: (0, i, 0)),
        ],
        out_specs=pl.BlockSpec((BN, D_OUT), lambda i: (i, 0)),
        out_shape=jax.ShapeDtypeStruct((N, D_OUT), jnp.float32),
    )
    hfin = tc2(Hs2, aggp2, degp)

    W1a = W1[:D_OUT]
    W1b = W1[D_OUT:]
    w3r = W3.reshape(1, -1)
    S = 5
    E_S = E // S
    BE = 3200
    grid_e = E_S // BE
    tc3 = pl.pallas_call(
        _tc3_body,
        grid=(grid_e,),
        in_specs=[
            pl.BlockSpec((BE, D_OUT), lambda i: (i, 0)),
            pl.BlockSpec((BE, D_OUT), lambda i: (i, 0)),
            pl.BlockSpec((D_OUT, D_HID), lambda i: (0, 0)),
            pl.BlockSpec((D_OUT, D_HID), lambda i: (0, 0)),
            pl.BlockSpec((1, D_HID), lambda i: (0, 0)),
            pl.BlockSpec((D_HID, D_HID), lambda i: (0, 0)),
            pl.BlockSpec((1, D_HID), lambda i: (0, 0)),
            pl.BlockSpec((1, D_HID), lambda i: (0, 0)),
            pl.BlockSpec((1, 1), lambda i: (0, 0)),
        ],
        out_specs=pl.BlockSpec((BE, 1), lambda i: (i, 0)),
        out_shape=jax.ShapeDtypeStruct((E_S, 1), jnp.float32),
    )
    scores = []
    prev_gu = None
    for s in range(S):
        pg = _make_pair_gather(N, D_OUT, E_S, offset=s * E_S)
        table = hfin
        if prev_gu is not None:
            table = table + 0.0 * prev_gu[0:1, 0:1]
        if len(scores) >= 2:
            table = table + 0.0 * scores[-2][0:1, 0:1]
        Gu, Gv = pg(table, src, dst)
        prev_gu = Gu
        scores.append(tc3(Gu, Gv, W1a, W1b, b1.reshape(1, -1), W2,
                          b2.reshape(1, -1), w3r, b3.reshape(1, 1)))
    return jnp.concatenate(scores, axis=0)


# speedup vs baseline: 7.9611x; 1.1527x over previous
import jax
import jax.numpy as jnp
from jax import lax
from jax.experimental import pallas as pl
from jax.experimental.pallas import tpu as pltpu
from jax.experimental.pallas import tpu_sc as plsc


_C = 40
_CP = 80
_K = 5


def _make_segsum(N, D, E, with_deg):
    info = plsc.get_sparse_core_info()
    NC, NS = info.num_cores, info.num_subcores
    NW = NC * NS
    EPW = E // NW
    C, K = _C, _K
    ITERS = EPW // C
    NR = ITERS // K
    RPS = (N // NS) // 8 * 8
    TAIL = N - RPS * NS
    mesh = plsc.VectorSubcoreMesh(core_axis_name="c", subcore_axis_name="s")

    if with_deg:
        out_type = [jax.ShapeDtypeStruct((NC, N, D), jnp.float32),
                    jax.ShapeDtypeStruct((NC, N, D), jnp.float32)]
    else:
        out_type = jax.ShapeDtypeStruct((NC, N, D), jnp.float32)
    scratch = [
        pltpu.VMEM((EPW,), jnp.int32),
        pltpu.VMEM((K, C), jnp.int32),
        pltpu.VMEM((K, C, D), jnp.float32),
        pltpu.VMEM_SHARED((N, D), jnp.float32),
    ] + [pltpu.SemaphoreType.DMA] * (3 * K)

    def body(table_hbm, src_hbm, dst_hbm, zeros_hbm, ones_hbm, *refs):
        if with_deg:
            agg_out, deg_out, sidx, didx, rows, acc_sh = refs[:6]
            sems = refs[6:]
        else:
            agg_out, sidx, didx, rows, acc_sh = refs[:5]
            sems = refs[5:]
        gsems, ssems, dsems = sems[:K], sems[K:2 * K], sems[2 * K:]
        cid = lax.axis_index("c")
        sid = lax.axis_index("s")
        wid = sid * NC + cid
        ebase = wid * EPW

        def copy_out(dst_hbm_arr):
            sl = pl.ds(sid * RPS, RPS)
            pltpu.sync_copy(acc_sh.at[sl], dst_hbm_arr.at[cid, sl])
            if TAIL:
                @pl.when(sid == NS - 1)
                def _tail():
                    tl = pl.ds(RPS * NS, TAIL)
                    pltpu.sync_copy(acc_sh.at[tl], dst_hbm_arr.at[cid, tl])

        def wait_rows(buf_ref, sem):
            pltpu.make_async_copy(table_hbm.at[pl.ds(0, C)], buf_ref,
                                  sem).wait()

        def wait_idx(buf_ref, sem):
            pltpu.make_async_copy(dst_hbm.at[pl.ds(0, C)], buf_ref,
                                  sem).wait()

        def load_didx(k, c):
            pltpu.async_copy(dst_hbm.at[pl.ds(ebase + c * C, C)],
                             didx.at[k], dsems[k])

        @pl.when(sid == 0)
        def _init():
            pltpu.sync_copy(zeros_hbm, acc_sh)

        pltpu.sync_copy(src_hbm.at[pl.ds(ebase, EPW)], sidx)
        plsc.subcore_barrier()

        for k in range(K):
            load_didx(k, k)
            pltpu.async_copy(table_hbm.at[sidx.at[pl.ds(k * C, C)]],
                             rows.at[k], gsems[k])

        def rnd(r, carry):
            for k in range(K):
                wait_rows(rows.at[k], gsems[k])
                wait_idx(didx.at[k], dsems[k])
                pltpu.async_copy(rows.at[k], acc_sh.at[didx.at[k]],
                                 ssems[k], add=True)
            for k in range(K):
                c = r * K + k
                wait_rows(rows.at[k], ssems[k])

                @pl.when(r < NR - 1)
                def _refire():
                    load_didx(k, c + K)
                    pltpu.async_copy(
                        table_hbm.at[sidx.at[pl.ds((c + K) * C, C)]],
                        rows.at[k], gsems[k])
            return carry

        lax.fori_loop(0, NR, rnd, 0)
        plsc.subcore_barrier()
        copy_out(agg_out)

        if with_deg:
            plsc.subcore_barrier()

            @pl.when(sid == 0)
            def _reinit():
                pltpu.sync_copy(zeros_hbm, acc_sh)

            pltpu.sync_copy(ones_hbm, rows.at[0])
            plsc.subcore_barrier()

            for k in range(K):
                load_didx(k, k)

            def drnd(r, carry):
                for k in range(K):
                    wait_idx(didx.at[k], dsems[k])
                    pltpu.async_copy(rows.at[0], acc_sh.at[didx.at[k]],
                                     ssems[k], add=True)
                for k in range(K):
                    c = r * K + k
                    wait_rows(rows.at[0], ssems[k])

                    @pl.when(r < NR - 1)
                    def _refire():
                        load_didx(k, c + K)
                return carry

            lax.fori_loop(0, NR, drnd, 0)
            plsc.subcore_barrier()
            copy_out(deg_out)

    return pl.kernel(body, mesh=mesh, out_type=out_type,
                     scratch_types=scratch)


def _make_pair_gather(N, D, E, offset=0):
    info = plsc.get_sparse_core_info()
    NC, NS = info.num_cores, info.num_subcores
    NW = NC * NS
    EPW = E // NW
    C, K = _CP, _K
    ITERS = EPW // C
    NR = ITERS // K
    mesh = plsc.VectorSubcoreMesh(core_axis_name="c", subcore_axis_name="s")

    out_type = [jax.ShapeDtypeStruct((E, D), jnp.float32),
                jax.ShapeDtypeStruct((E, D), jnp.float32)]
    scratch = [
        pltpu.VMEM((EPW,), jnp.int32),
        pltpu.VMEM((EPW,), jnp.int32),
        pltpu.VMEM((K, C, D), jnp.float32),
        pltpu.VMEM((K, C, D), jnp.float32),
    ] + [pltpu.SemaphoreType.DMA] * (4 * K)

    def body(table_hbm, src_hbm, dst_hbm, gu_out, gv_out, *refs):
        sidx, didx, ru, rv = refs[:4]
        gusems = refs[4:4 + K]
        gvsems = refs[4 + K:4 + 2 * K]
        wusems = refs[4 + 2 * K:4 + 3 * K]
        wvsems = refs[4 + 3 * K:]
        cid = lax.axis_index("c")
        sid = lax.axis_index("s")
        wid = sid * NC + cid
        ebase = wid * EPW

        def wait(buf_ref, sem):
            pltpu.make_async_copy(table_hbm.at[pl.ds(0, C)], buf_ref,
                                  sem).wait()

        pltpu.sync_copy(src_hbm.at[pl.ds(offset + ebase, EPW)], sidx)
        pltpu.sync_copy(dst_hbm.at[pl.ds(offset + ebase, EPW)], didx)

        def fire(k, c):
            pltpu.async_copy(table_hbm.at[sidx.at[pl.ds(c * C, C)]],
                             ru.at[k], gusems[k])
            pltpu.async_copy(table_hbm.at[didx.at[pl.ds(c * C, C)]],
                             rv.at[k], gvsems[k])

        for k in range(K):
            fire(k, k)

        def rnd(r, carry):
            for k in range(K):
                c = r * K + k
                base = ebase + c * C
                wait(ru.at[k], gusems[k])
                pltpu.async_copy(ru.at[k], gu_out.at[pl.ds(base, C)],
                                 wusems[k])
                wait(rv.at[k], gvsems[k])
                pltpu.async_copy(rv.at[k], gv_out.at[pl.ds(base, C)],
                                 wvsems[k])
            for k in range(K):
                c = r * K + k
                wait(ru.at[k], wusems[k])
                wait(rv.at[k], wvsems[k])

                @pl.when(r < NR - 1)
                def _refire():
                    fire(k, c + K)
            return carry

        lax.fori_loop(0, NR, rnd, 0)

    return pl.kernel(body, mesh=mesh, out_type=out_type,
                     scratch_types=scratch)


def _tc1_body(x_ref, aggp_ref, degp_ref, ws1_ref, bs1_ref, wn1_ref,
              ws2_ref, bs2_ref, wn2_ref, m2_ref, hs2_ref):
    agg = aggp_ref[0] + aggp_ref[1]
    deg = degp_ref[0, :, 0] + degp_ref[1, :, 0]
    inv = 1.0 / jnp.maximum(deg, 1.0)
    hn = agg * inv[:, None]
    h = x_ref[...] @ ws1_ref[...] + bs1_ref[...] + hn @ wn1_ref[...]
    h = jnp.maximum(h, 0.0)
    m2_ref[...] = h @ wn2_ref[...]
    hs2_ref[...] = h @ ws2_ref[...] + bs2_ref[...]


def _tc2_body(hs2_ref, aggp_ref, degp_ref, hfin_ref):
    agg = aggp_ref[0] + aggp_ref[1]
    deg = degp_ref[0, :, 0] + degp_ref[1, :, 0]
    inv = 1.0 / jnp.maximum(deg, 1.0)
    hfin_ref[...] = hs2_ref[...] + agg * inv[:, None]


def _tc3_body(gu_ref, gv_ref, w1a_ref, w1b_ref, b1_ref, w2_ref, b2_ref,
              w3_ref, b3_ref, out_ref):
    z = gu_ref[...] @ w1a_ref[...] + gv_ref[...] @ w1b_ref[...] + b1_ref[...]
    z = jnp.maximum(z, 0.0)
    z = jnp.maximum(z @ w2_ref[...] + b2_ref[...], 0.0)
    s = jnp.sum(z * w3_ref[...], axis=1) + b3_ref[0, 0]
    i = pl.program_id(0)
    rows = s.shape[0] // 128
    out_ref[pl.ds(i * rows, rows), :] = jax.nn.sigmoid(s).reshape(rows, 128)


def kernel(x, edge_index, W_self1, b_self1, W_neigh1, W_self2, b_self2,
           W_neigh2, W1, b1, W2, b2, W3, b3):
    N, D_IN = x.shape
    E = edge_index.shape[1]
    D_HID = W_self1.shape[1]
    D_OUT = W_self2.shape[1]
    src = edge_index[0].astype(jnp.int32)
    dst = edge_index[1].astype(jnp.int32)

    info = plsc.get_sparse_core_info()
    NC = info.num_cores

    zeros_n = jnp.zeros((N, D_IN), jnp.float32)
    ones_c = jnp.ones((_C, D_IN), jnp.float32)
    segsum1 = _make_segsum(N, D_IN, E, with_deg=True)
    aggp1, degp = segsum1(x, src, dst, zeros_n, ones_c)

    BN = 1000
    grid_n = N // BN
    tc1 = pl.pallas_call(
        _tc1_body,
        grid=(grid_n,),
        in_specs=[
            pl.BlockSpec((BN, D_IN), lambda i: (i, 0)),
            pl.BlockSpec((NC, BN, D_IN), lambda i: (0, i, 0)),
            pl.BlockSpec((NC, BN, D_IN), lambda i: (0, i, 0)),
            pl.BlockSpec((D_IN, D_HID), lambda i: (0, 0)),
            pl.BlockSpec((1, D_HID), lambda i: (0, 0)),
            pl.BlockSpec((D_IN, D_HID), lambda i: (0, 0)),
            pl.BlockSpec((D_HID, D_OUT), lambda i: (0, 0)),
            pl.BlockSpec((1, D_OUT), lambda i: (0, 0)),
            pl.BlockSpec((D_HID, D_OUT), lambda i: (0, 0)),
        ],
        out_specs=[
            pl.BlockSpec((BN, D_OUT), lambda i: (i, 0)),
            pl.BlockSpec((BN, D_OUT), lambda i: (i, 0)),
        ],
        out_shape=[
            jax.ShapeDtypeStruct((N, D_OUT), jnp.float32),
            jax.ShapeDtypeStruct((N, D_OUT), jnp.float32),
        ],
    )
    M2, Hs2 = tc1(x, aggp1, degp, W_self1, b_self1.reshape(1, -1), W_neigh1,
                  W_self2, b_self2.reshape(1, -1), W_neigh2)

    segsum2 = _make_segsum(N, D_OUT, E, with_deg=False)
    aggp2 = segsum2(M2, src, dst, zeros_n, ones_c)

    tc2 = pl.pallas_call(
        _tc2_body,
        grid=(grid_n,),
        in_specs=[
            pl.BlockSpec((BN, D_OUT), lambda i: (i, 0)),
            pl.BlockSpec((NC, BN, D_OUT), lambda i: (0, i, 0)),
            pl.BlockSpec((NC, BN, D_IN), lambda i: (0, i, 0)),
        ],
        out_specs=pl.BlockSpec((BN, D_OUT), lambda i: (i, 0)),
        out_shape=jax.ShapeDtypeStruct((N, D_OUT), jnp.float32),
    )
    hfin = tc2(Hs2, aggp2, degp)

    W1a = W1[:D_OUT]
    W1b = W1[D_OUT:]
    w3r = W3.reshape(1, -1)
    S = 5
    E_S = E // S
    BE = 3200
    grid_e = E_S // BE
    tc3 = pl.pallas_call(
        _tc3_body,
        grid=(grid_e,),
        in_specs=[
            pl.BlockSpec((BE, D_OUT), lambda i: (i, 0)),
            pl.BlockSpec((BE, D_OUT), lambda i: (i, 0)),
            pl.BlockSpec((D_OUT, D_HID), lambda i: (0, 0)),
            pl.BlockSpec((D_OUT, D_HID), lambda i: (0, 0)),
            pl.BlockSpec((1, D_HID), lambda i: (0, 0)),
            pl.BlockSpec((D_HID, D_HID), lambda i: (0, 0)),
            pl.BlockSpec((1, D_HID), lambda i: (0, 0)),
            pl.BlockSpec((1, D_HID), lambda i: (0, 0)),
            pl.BlockSpec((1, 1), lambda i: (0, 0)),
        ],
        out_specs=pl.BlockSpec((E_S // 128, 128), lambda i: (0, 0)),
        out_shape=jax.ShapeDtypeStruct((E_S // 128, 128), jnp.float32),
    )
    scores = []
    prev_gu = None
    for s in range(S):
        pg = _make_pair_gather(N, D_OUT, E_S, offset=s * E_S)
        table = hfin
        if prev_gu is not None:
            table = table + 0.0 * prev_gu[0:1, 0:1]
        if len(scores) >= 2:
            table = table + 0.0 * scores[-2][0, 0]
        Gu, Gv = pg(table, src, dst)
        prev_gu = Gu
        scores.append(tc3(Gu, Gv, W1a, W1b, b1.reshape(1, -1), W2,
                          b2.reshape(1, -1), w3r, b3.reshape(1, 1)))
    return jnp.concatenate(scores, axis=0).reshape(E, 1)


# speedup vs baseline: 8.1195x; 1.0199x over previous
import jax
import jax.numpy as jnp
from jax import lax
from jax.experimental import pallas as pl
from jax.experimental.pallas import tpu as pltpu
from jax.experimental.pallas import tpu_sc as plsc


_C = 40
_CP = 80
_K = 5


def _make_segsum(N, D, E, with_deg):
    info = plsc.get_sparse_core_info()
    NC, NS = info.num_cores, info.num_subcores
    NW = NC * NS
    EPW = E // NW
    C, K = _C, _K
    ITERS = EPW // C
    NR = ITERS // K
    RPS = (N // NS) // 8 * 8
    TAIL = N - RPS * NS
    mesh = plsc.VectorSubcoreMesh(core_axis_name="c", subcore_axis_name="s")

    if with_deg:
        out_type = [jax.ShapeDtypeStruct((NC, N, D), jnp.float32),
                    jax.ShapeDtypeStruct((NC, N, D), jnp.float32)]
    else:
        out_type = jax.ShapeDtypeStruct((NC, N, D), jnp.float32)
    scratch = [
        pltpu.VMEM((EPW,), jnp.int32),
        pltpu.VMEM((K, C), jnp.int32),
        pltpu.VMEM((K, C, D), jnp.float32),
        pltpu.VMEM_SHARED((N, D), jnp.float32),
    ] + [pltpu.SemaphoreType.DMA] * (3 * K)

    def body(table_hbm, src_hbm, dst_hbm, zeros_hbm, ones_hbm, *refs):
        if with_deg:
            agg_out, deg_out, sidx, didx, rows, acc_sh = refs[:6]
            sems = refs[6:]
        else:
            agg_out, sidx, didx, rows, acc_sh = refs[:5]
            sems = refs[5:]
        gsems, ssems, dsems = sems[:K], sems[K:2 * K], sems[2 * K:]
        cid = lax.axis_index("c")
        sid = lax.axis_index("s")
        wid = sid * NC + cid
        ebase = wid * EPW

        def copy_out(dst_hbm_arr):
            sl = pl.ds(sid * RPS, RPS)
            pltpu.sync_copy(acc_sh.at[sl], dst_hbm_arr.at[cid, sl])
            if TAIL:
                @pl.when(sid == NS - 1)
                def _tail():
                    tl = pl.ds(RPS * NS, TAIL)
                    pltpu.sync_copy(acc_sh.at[tl], dst_hbm_arr.at[cid, tl])

        def wait_rows(buf_ref, sem):
            pltpu.make_async_copy(table_hbm.at[pl.ds(0, C)], buf_ref,
                                  sem).wait()

        def wait_idx(buf_ref, sem):
            pltpu.make_async_copy(dst_hbm.at[pl.ds(0, C)], buf_ref,
                                  sem).wait()

        def load_didx(k, c):
            pltpu.async_copy(dst_hbm.at[pl.ds(ebase + c * C, C)],
                             didx.at[k], dsems[k])

        @pl.when(sid == 0)
        def _init():
            pltpu.sync_copy(zeros_hbm, acc_sh)

        pltpu.sync_copy(src_hbm.at[pl.ds(ebase, EPW)], sidx)
        plsc.subcore_barrier()

        for k in range(K):
            load_didx(k, k)
            pltpu.async_copy(table_hbm.at[sidx.at[pl.ds(k * C, C)]],
                             rows.at[k], gsems[k])

        def rnd(r, carry):
            for k in range(K):
                wait_rows(rows.at[k], gsems[k])
                wait_idx(didx.at[k], dsems[k])
                pltpu.async_copy(rows.at[k], acc_sh.at[didx.at[k]],
                                 ssems[k], add=True)
            for k in range(K):
                c = r * K + k
                wait_rows(rows.at[k], ssems[k])

                @pl.when(r < NR - 1)
                def _refire():
                    load_didx(k, c + K)
                    pltpu.async_copy(
                        table_hbm.at[sidx.at[pl.ds((c + K) * C, C)]],
                        rows.at[k], gsems[k])
            return carry

        lax.fori_loop(0, NR, rnd, 0)
        plsc.subcore_barrier()
        copy_out(agg_out)

        if with_deg:
            plsc.subcore_barrier()

            @pl.when(sid == 0)
            def _reinit():
                pltpu.sync_copy(zeros_hbm, acc_sh)

            pltpu.sync_copy(ones_hbm, rows.at[0])
            plsc.subcore_barrier()

            for k in range(K):
                load_didx(k, k)

            def drnd(r, carry):
                for k in range(K):
                    wait_idx(didx.at[k], dsems[k])
                    pltpu.async_copy(rows.at[0], acc_sh.at[didx.at[k]],
                                     ssems[k], add=True)
                for k in range(K):
                    c = r * K + k
                    wait_rows(rows.at[0], ssems[k])

                    @pl.when(r < NR - 1)
                    def _refire():
                        load_didx(k, c + K)
                return carry

            lax.fori_loop(0, NR, drnd, 0)
            plsc.subcore_barrier()
            copy_out(deg_out)

    return pl.kernel(body, mesh=mesh, out_type=out_type,
                     scratch_types=scratch)


def _make_pair_gather(N, D, E, offset=0):
    info = plsc.get_sparse_core_info()
    NC, NS = info.num_cores, info.num_subcores
    NW = NC * NS
    EPW = E // NW
    C, K = _CP, _K
    ITERS = EPW // C
    NR = ITERS // K
    mesh = plsc.VectorSubcoreMesh(core_axis_name="c", subcore_axis_name="s")

    out_type = [jax.ShapeDtypeStruct((E, D), jnp.float32),
                jax.ShapeDtypeStruct((E, D), jnp.float32)]
    scratch = [
        pltpu.VMEM((EPW,), jnp.int32),
        pltpu.VMEM((EPW,), jnp.int32),
        pltpu.VMEM((K, C, D), jnp.float32),
        pltpu.VMEM((K, C, D), jnp.float32),
    ] + [pltpu.SemaphoreType.DMA] * (4 * K)

    def body(table_hbm, src_hbm, dst_hbm, gu_out, gv_out, *refs):
        sidx, didx, ru, rv = refs[:4]
        gusems = refs[4:4 + K]
        gvsems = refs[4 + K:4 + 2 * K]
        wusems = refs[4 + 2 * K:4 + 3 * K]
        wvsems = refs[4 + 3 * K:]
        cid = lax.axis_index("c")
        sid = lax.axis_index("s")
        wid = sid * NC + cid
        ebase = wid * EPW

        def wait(buf_ref, sem):
            pltpu.make_async_copy(table_hbm.at[pl.ds(0, C)], buf_ref,
                                  sem).wait()

        pltpu.sync_copy(src_hbm.at[pl.ds(offset + ebase, EPW)], sidx)
        pltpu.sync_copy(dst_hbm.at[pl.ds(offset + ebase, EPW)], didx)

        def fire(k, c):
            pltpu.async_copy(table_hbm.at[sidx.at[pl.ds(c * C, C)]],
                             ru.at[k], gusems[k])
            pltpu.async_copy(table_hbm.at[didx.at[pl.ds(c * C, C)]],
                             rv.at[k], gvsems[k])

        for k in range(K):
            fire(k, k)

        def rnd(r, carry):
            for k in range(K):
                c = r * K + k
                base = ebase + c * C
                wait(ru.at[k], gusems[k])
                pltpu.async_copy(ru.at[k], gu_out.at[pl.ds(base, C)],
                                 wusems[k])
                wait(rv.at[k], gvsems[k])
                pltpu.async_copy(rv.at[k], gv_out.at[pl.ds(base, C)],
                                 wvsems[k])
            for k in range(K):
                c = r * K + k
                wait(ru.at[k], wusems[k])
                wait(rv.at[k], wvsems[k])

                @pl.when(r < NR - 1)
                def _refire():
                    fire(k, c + K)
            return carry

        lax.fori_loop(0, NR, rnd, 0)

    return pl.kernel(body, mesh=mesh, out_type=out_type,
                     scratch_types=scratch)


def _tc1_body(x_ref, aggp_ref, degp_ref, ws1_ref, bs1_ref, wn1_ref,
              ws2_ref, bs2_ref, wn2_ref, m2_ref, hs2_ref):
    agg = aggp_ref[0] + aggp_ref[1]
    deg = degp_ref[0, :, 0] + degp_ref[1, :, 0]
    inv = 1.0 / jnp.maximum(deg, 1.0)
    hn = agg * inv[:, None]
    h = x_ref[...] @ ws1_ref[...] + bs1_ref[...] + hn @ wn1_ref[...]
    h = jnp.maximum(h, 0.0)
    m2_ref[...] = h @ wn2_ref[...]
    hs2_ref[...] = h @ ws2_ref[...] + bs2_ref[...]


def _tc2_body(hs2_ref, aggp_ref, degp_ref, hfin_ref):
    agg = aggp_ref[0] + aggp_ref[1]
    deg = degp_ref[0, :, 0] + degp_ref[1, :, 0]
    inv = 1.0 / jnp.maximum(deg, 1.0)
    hfin_ref[...] = hs2_ref[...] + agg * inv[:, None]


def _tc3_body(gu_ref, gv_ref, w1a_ref, w1b_ref, b1_ref, w2_ref, b2_ref,
              w3_ref, b3_ref, out_ref):
    z = gu_ref[...] @ w1a_ref[...] + gv_ref[...] @ w1b_ref[...] + b1_ref[...]
    z = jnp.maximum(z, 0.0)
    z = jnp.maximum(z @ w2_ref[...] + b2_ref[...], 0.0)
    s = jnp.sum(z * w3_ref[...], axis=1) + b3_ref[0, 0]
    i = pl.program_id(0)
    rows = s.shape[0] // 128
    out_ref[pl.ds(i * rows, rows), :] = jax.nn.sigmoid(s).reshape(rows, 128)


def kernel(x, edge_index, W_self1, b_self1, W_neigh1, W_self2, b_self2,
           W_neigh2, W1, b1, W2, b2, W3, b3):
    N, D_IN = x.shape
    E = edge_index.shape[1]
    D_HID = W_self1.shape[1]
    D_OUT = W_self2.shape[1]
    src = edge_index[0].astype(jnp.int32)
    dst = edge_index[1].astype(jnp.int32)

    info = plsc.get_sparse_core_info()
    NC = info.num_cores

    zeros_n = jnp.zeros((N, D_IN), jnp.float32)
    ones_c = jnp.ones((_C, D_IN), jnp.float32)
    segsum1 = _make_segsum(N, D_IN, E, with_deg=True)
    aggp1, degp = segsum1(x, src, dst, zeros_n, ones_c)

    BN = 1000
    grid_n = N // BN
    tc1 = pl.pallas_call(
        _tc1_body,
        grid=(grid_n,),
        in_specs=[
            pl.BlockSpec((BN, D_IN), lambda i: (i, 0)),
            pl.BlockSpec((NC, BN, D_IN), lambda i: (0, i, 0)),
            pl.BlockSpec((NC, BN, D_IN), lambda i: (0, i, 0)),
            pl.BlockSpec((D_IN, D_HID), lambda i: (0, 0)),
            pl.BlockSpec((1, D_HID), lambda i: (0, 0)),
            pl.BlockSpec((D_IN, D_HID), lambda i: (0, 0)),
            pl.BlockSpec((D_HID, D_OUT), lambda i: (0, 0)),
            pl.BlockSpec((1, D_OUT), lambda i: (0, 0)),
            pl.BlockSpec((D_HID, D_OUT), lambda i: (0, 0)),
        ],
        out_specs=[
            pl.BlockSpec((BN, D_OUT), lambda i: (i, 0)),
            pl.BlockSpec((BN, D_OUT), lambda i: (i, 0)),
        ],
        out_shape=[
            jax.ShapeDtypeStruct((N, D_OUT), jnp.float32),
            jax.ShapeDtypeStruct((N, D_OUT), jnp.float32),
        ],
    )
    M2, Hs2 = tc1(x, aggp1, degp, W_self1, b_self1.reshape(1, -1), W_neigh1,
                  W_self2, b_self2.reshape(1, -1), W_neigh2)

    segsum2 = _make_segsum(N, D_OUT, E, with_deg=False)
    aggp2 = segsum2(M2, src, dst, zeros_n, ones_c)

    tc2 = pl.pallas_call(
        _tc2_body,
        grid=(grid_n,),
        in_specs=[
            pl.BlockSpec((BN, D_OUT), lambda i: (i, 0)),
            pl.BlockSpec((NC, BN, D_OUT), lambda i: (0, i, 0)),
            pl.BlockSpec((NC, BN, D_IN), lambda i: (0, i, 0)),
        ],
        out_specs=pl.BlockSpec((BN, D_OUT), lambda i: (i, 0)),
        out_shape=jax.ShapeDtypeStruct((N, D_OUT), jnp.float32),
    )
    hfin = tc2(Hs2, aggp2, degp)

    W1a = W1[:D_OUT]
    W1b = W1[D_OUT:]
    w3r = W3.reshape(1, -1)
    S = 5
    E_S = E // S
    BE = 6400
    grid_e = E_S // BE
    tc3 = pl.pallas_call(
        _tc3_body,
        grid=(grid_e,),
        in_specs=[
            pl.BlockSpec((BE, D_OUT), lambda i: (i, 0)),
            pl.BlockSpec((BE, D_OUT), lambda i: (i, 0)),
            pl.BlockSpec((D_OUT, D_HID), lambda i: (0, 0)),
            pl.BlockSpec((D_OUT, D_HID), lambda i: (0, 0)),
            pl.BlockSpec((1, D_HID), lambda i: (0, 0)),
            pl.BlockSpec((D_HID, D_HID), lambda i: (0, 0)),
            pl.BlockSpec((1, D_HID), lambda i: (0, 0)),
            pl.BlockSpec((1, D_HID), lambda i: (0, 0)),
            pl.BlockSpec((1, 1), lambda i: (0, 0)),
        ],
        out_specs=pl.BlockSpec((E_S // 128, 128), lambda i: (0, 0)),
        out_shape=jax.ShapeDtypeStruct((E_S // 128, 128), jnp.float32),
    )
    scores = []
    prev_gu = None
    for s in range(S):
        pg = _make_pair_gather(N, D_OUT, E_S, offset=s * E_S)
        table = hfin
        if prev_gu is not None:
            table = table + 0.0 * prev_gu[0:1, 0:1]
        if len(scores) >= 2:
            table = table + 0.0 * scores[-2][0, 0]
        Gu, Gv = pg(table, src, dst)
        prev_gu = Gu
        scores.append(tc3(Gu, Gv, W1a, W1b, b1.reshape(1, -1), W2,
                          b2.reshape(1, -1), w3r, b3.reshape(1, 1)))
    return jnp.concatenate(scores, axis=0).reshape(E, 1)
